# Initial kernel scaffold; baseline (speedup 1.0000x reference)
import functools

import jax
import jax.numpy as jnp
from jax import lax
from jax.experimental import pallas as pl
from jax.experimental.pallas import tpu as pltpu
from jax.experimental.pallas import tpu_sc as plsc

EMB = 32
HID = 128

_FIRE = 128
_FPG = 8
_GROUP = _FIRE * _FPG


def _sc_gather(x2d, w):
    n = x2d.shape[0] * x2d.shape[1]
    info = plsc.get_sparse_core_info()
    nc, ns = info.num_cores, info.num_subcores
    nw = nc * ns
    per_w = n // nw
    groups = per_w // _GROUP

    mesh = plsc.VectorSubcoreMesh(core_axis_name="c", subcore_axis_name="s")

    @functools.partial(
        pl.kernel,
        mesh=mesh,
        out_type=jax.ShapeDtypeStruct((n, EMB), jnp.float32),
        scratch_types=[
            pltpu.VMEM((_FPG, _FIRE), jnp.int32),
            pltpu.VMEM((_GROUP, EMB), jnp.float32),
            pltpu.SemaphoreType.DMA,
        ],
    )
    def gather(x_hbm, w_hbm, out_hbm, idx_v, rows_v, sem):
        wid = lax.axis_index("s") * nc + lax.axis_index("c")
        row_base = wid * (per_w // _FIRE)

        def body(g, carry):
            pltpu.sync_copy(x_hbm.at[pl.ds(row_base + g * _FPG, _FPG)], idx_v)
            copies = []
            for j in range(_FPG):
                copies.append(
                    pltpu.async_copy(
                        w_hbm.at[idx_v.at[j]],
                        rows_v.at[pl.ds(j * _FIRE, _FIRE)],
                        sem,
                    )
                )
            for c in copies:
                c.wait()
            out_off = wid * per_w + g * _GROUP
            pltpu.sync_copy(rows_v, out_hbm.at[pl.ds(out_off, _GROUP)])
            return carry

        lax.fori_loop(0, groups, body, 0)

    return gather(x2d, w)


def _tc_project(g, we):
    n = g.shape[0]
    bm = 4096

    def mm(g_ref, we_ref, o_ref):
        o_ref[...] = lax.dot_general(
            g_ref[...],
            we_ref[...],
            (((1,), (1,)), ((), ())),
            preferred_element_type=jnp.float32,
        )

    return pl.pallas_call(
        mm,
        grid=(n // bm,),
        in_specs=[
            pl.BlockSpec((bm, EMB), lambda i: (i, 0)),
            pl.BlockSpec((HID, EMB), lambda i: (0, 0)),
        ],
        out_specs=pl.BlockSpec((bm, HID), lambda i: (i, 0)),
        out_shape=jax.ShapeDtypeStruct((n, HID), jnp.float32),
    )(g, we)


def kernel(x, W, We):
    b, l = x.shape
    n = b * l
    x2d = x.reshape(n // _FIRE, _FIRE).astype(jnp.int32)
    g = _sc_gather(x2d, W)
    out = _tc_project(g, We)
    return out.reshape(b, l, HID)


# speedup vs baseline: 17.8346x; 17.8346x over previous
import functools

import jax
import jax.numpy as jnp
from jax import lax
from jax.experimental import pallas as pl
from jax.experimental.pallas import tpu as pltpu
from jax.experimental.pallas import tpu_sc as plsc

EMB = 32
HID = 128

_FIRE = 128
_FPG = 8
_GROUP = _FIRE * _FPG


def _sc_gather(x2d, w):
    n = x2d.shape[0] * x2d.shape[1]
    info = plsc.get_sparse_core_info()
    nc, ns = info.num_cores, info.num_subcores
    nw = nc * ns
    per_w = n // nw
    groups = per_w // _GROUP

    mesh = plsc.VectorSubcoreMesh(core_axis_name="c", subcore_axis_name="s")

    @functools.partial(
        pl.kernel,
        mesh=mesh,
        out_type=jax.ShapeDtypeStruct((n, EMB), jnp.float32),
        scratch_types=[
            pltpu.VMEM((_FPG, _FIRE), jnp.int32),
            pltpu.VMEM((_GROUP, EMB), jnp.float32),
            pltpu.SemaphoreType.DMA,
        ],
        compiler_params=pltpu.CompilerParams(use_tc_tiling_on_sc=False),
    )
    def gather(x_hbm, w_hbm, out_hbm, idx_v, rows_v, sem):
        wid = lax.axis_index("s") * nc + lax.axis_index("c")
        row_base = wid * (per_w // _FIRE)

        def body(g, carry):
            pltpu.sync_copy(x_hbm.at[pl.ds(row_base + g * _FPG, _FPG)], idx_v)
            copies = []
            for j in range(_FPG):
                copies.append(
                    pltpu.async_copy(
                        w_hbm.at[idx_v.at[j]],
                        rows_v.at[pl.ds(j * _FIRE, _FIRE)],
                        sem,
                    )
                )
            for c in copies:
                c.wait()
            out_off = wid * per_w + g * _GROUP
            pltpu.sync_copy(rows_v, out_hbm.at[pl.ds(out_off, _GROUP)])
            return carry

        lax.fori_loop(0, groups, body, 0)

    return gather(x2d, w)


def _tc_project(g, we):
    n = g.shape[0]
    bm = 4096

    def mm(g_ref, we_ref, o_ref):
        o_ref[...] = lax.dot_general(
            g_ref[...],
            we_ref[...],
            (((1,), (1,)), ((), ())),
            preferred_element_type=jnp.float32,
        )

    return pl.pallas_call(
        mm,
        grid=(n // bm,),
        in_specs=[
            pl.BlockSpec((bm, EMB), lambda i: (i, 0)),
            pl.BlockSpec((HID, EMB), lambda i: (0, 0)),
        ],
        out_specs=pl.BlockSpec((bm, HID), lambda i: (i, 0)),
        out_shape=jax.ShapeDtypeStruct((n, HID), jnp.float32),
    )(g, we)


def kernel(x, W, We):
    b, l = x.shape
    n = b * l
    x2d = x.reshape(n // _FIRE, _FIRE).astype(jnp.int32)
    g = _sc_gather(x2d, W)
    out = _tc_project(g, We)
    return out.reshape(b, l, HID)


# speedup vs baseline: 18.1834x; 1.0196x over previous
import functools

import jax
import jax.numpy as jnp
from jax import lax
from jax.experimental import pallas as pl
from jax.experimental.pallas import tpu as pltpu
from jax.experimental.pallas import tpu_sc as plsc

EMB = 32
HID = 128

_FIRE = 128
_FPG = 8
_GROUP = _FIRE * _FPG


def _sc_gather(x2d, w):
    n = x2d.shape[0] * x2d.shape[1]
    info = plsc.get_sparse_core_info()
    nc, ns = info.num_cores, info.num_subcores
    nw = nc * ns
    per_w = n // nw
    groups = per_w // _GROUP

    mesh = plsc.VectorSubcoreMesh(core_axis_name="c", subcore_axis_name="s")

    @functools.partial(
        pl.kernel,
        mesh=mesh,
        out_type=jax.ShapeDtypeStruct((n, EMB), jnp.float32),
        scratch_types=[
            pltpu.VMEM((2, _FPG, _FIRE), jnp.int32),
            pltpu.VMEM((2, _GROUP, EMB), jnp.float32),
            pltpu.SemaphoreType.DMA,
            pltpu.SemaphoreType.DMA,
            pltpu.SemaphoreType.DMA,
        ],
        compiler_params=pltpu.CompilerParams(use_tc_tiling_on_sc=False),
    )
    def gather(x_hbm, w_hbm, out_hbm, idx_v, rows_v, sem_g, sem_i, sem_o):
        wid = lax.axis_index("s") * nc + lax.axis_index("c")
        row_base = wid * (per_w // _FIRE)
        out_base = wid * per_w

        def fire_group(slot):
            for j in range(_FPG):
                pltpu.async_copy(
                    w_hbm.at[idx_v.at[slot, j]],
                    rows_v.at[slot, pl.ds(j * _FIRE, _FIRE)],
                    sem_g,
                )

        pltpu.sync_copy(x_hbm.at[pl.ds(row_base, _FPG)], idx_v.at[0])
        fire_group(0)

        def body(g, carry):
            slot = lax.rem(g, 2)
            nslot = 1 - slot

            @pl.when(g + 1 < groups)
            def _():
                pltpu.async_copy(
                    x_hbm.at[pl.ds(row_base + (g + 1) * _FPG, _FPG)],
                    idx_v.at[nslot],
                    sem_i,
                )

            pltpu.make_async_copy(
                out_hbm.at[pl.ds(out_base, _GROUP)],
                rows_v.at[slot],
                sem_g,
            ).wait()

            @pl.when(g >= 1)
            def _():
                pltpu.make_async_copy(
                    rows_v.at[nslot],
                    out_hbm.at[pl.ds(out_base, _GROUP)],
                    sem_o,
                ).wait()

            @pl.when(g + 1 < groups)
            def _():
                pltpu.make_async_copy(
                    x_hbm.at[pl.ds(row_base, _FPG)],
                    idx_v.at[nslot],
                    sem_i,
                ).wait()
                fire_group(nslot)

            pltpu.async_copy(
                rows_v.at[slot],
                out_hbm.at[pl.ds(out_base + g * _GROUP, _GROUP)],
                sem_o,
            )
            return carry

        lax.fori_loop(0, groups, body, 0)

        pltpu.make_async_copy(
            rows_v.at[(groups - 1) % 2],
            out_hbm.at[pl.ds(out_base, _GROUP)],
            sem_o,
        ).wait()

    return gather(x2d, w)


def _tc_project(g, we):
    n = g.shape[0]
    bm = 4096

    def mm(g_ref, we_ref, o_ref):
        o_ref[...] = lax.dot_general(
            g_ref[...],
            we_ref[...],
            (((1,), (1,)), ((), ())),
            preferred_element_type=jnp.float32,
        )

    return pl.pallas_call(
        mm,
        grid=(n // bm,),
        in_specs=[
            pl.BlockSpec((bm, EMB), lambda i: (i, 0)),
            pl.BlockSpec((HID, EMB), lambda i: (0, 0)),
        ],
        out_specs=pl.BlockSpec((bm, HID), lambda i: (i, 0)),
        out_shape=jax.ShapeDtypeStruct((n, HID), jnp.float32),
    )(g, we)


def kernel(x, W, We):
    b, l = x.shape
    n = b * l
    x2d = x.reshape(n // _FIRE, _FIRE).astype(jnp.int32)
    g = _sc_gather(x2d, W)
    out = _tc_project(g, We)
    return out.reshape(b, l, HID)


# speedup vs baseline: 22.7700x; 1.2522x over previous
import functools

import jax
import jax.numpy as jnp
from jax import lax
from jax.experimental import pallas as pl
from jax.experimental.pallas import tpu as pltpu
from jax.experimental.pallas import tpu_sc as plsc

EMB = 32
HID = 128

_FIRE = 128
_FPG = 8
_GROUP = _FIRE * _FPG


def _sc_gather(x2d, w):
    n = x2d.shape[0] * x2d.shape[1]
    info = plsc.get_sparse_core_info()
    nc, ns = info.num_cores, info.num_subcores
    nw = nc * ns
    per_w = n // nw
    groups = per_w // _GROUP

    mesh = plsc.VectorSubcoreMesh(core_axis_name="c", subcore_axis_name="s")

    @functools.partial(
        pl.kernel,
        mesh=mesh,
        out_type=jax.ShapeDtypeStruct((n, HID), jnp.float32),
        scratch_types=[
            pltpu.VMEM((2, _FPG, _FIRE), jnp.int32),
            pltpu.VMEM((2, _GROUP, EMB), jnp.float32),
            pltpu.SemaphoreType.DMA,
            pltpu.SemaphoreType.DMA,
            pltpu.SemaphoreType.DMA,
        ],
        compiler_params=pltpu.CompilerParams(use_tc_tiling_on_sc=False),
    )
    def gather(x_hbm, w_hbm, out_hbm, idx_v, rows_v, sem_g, sem_i, sem_o):
        wid = lax.axis_index("s") * nc + lax.axis_index("c")
        row_base = wid * (per_w // _FIRE)
        out_base = wid * per_w

        def fire_group(slot):
            for j in range(_FPG):
                pltpu.async_copy(
                    w_hbm.at[idx_v.at[slot, j]],
                    rows_v.at[slot, pl.ds(j * _FIRE, _FIRE)],
                    sem_g,
                )

        pltpu.sync_copy(x_hbm.at[pl.ds(row_base, _FPG)], idx_v.at[0])
        fire_group(0)

        def body(g, carry):
            slot = lax.rem(g, 2)
            nslot = 1 - slot

            @pl.when(g + 1 < groups)
            def _():
                pltpu.async_copy(
                    x_hbm.at[pl.ds(row_base + (g + 1) * _FPG, _FPG)],
                    idx_v.at[nslot],
                    sem_i,
                )

            pltpu.make_async_copy(
                out_hbm.at[pl.ds(out_base, _GROUP), pl.ds(0, EMB)],
                rows_v.at[slot],
                sem_g,
            ).wait()

            @pl.when(g >= 1)
            def _():
                pltpu.make_async_copy(
                    rows_v.at[nslot],
                    out_hbm.at[pl.ds(out_base, _GROUP), pl.ds(0, EMB)],
                    sem_o,
                ).wait()

            @pl.when(g + 1 < groups)
            def _():
                pltpu.make_async_copy(
                    x_hbm.at[pl.ds(row_base, _FPG)],
                    idx_v.at[nslot],
                    sem_i,
                ).wait()
                fire_group(nslot)

            pltpu.async_copy(
                rows_v.at[slot],
                out_hbm.at[pl.ds(out_base + g * _GROUP, _GROUP), pl.ds(0, EMB)],
                sem_o,
            )
            return carry

        lax.fori_loop(0, groups, body, 0)

        pltpu.make_async_copy(
            rows_v.at[(groups - 1) % 2],
            out_hbm.at[pl.ds(out_base, _GROUP), pl.ds(0, EMB)],
            sem_o,
        ).wait()

    return gather(x2d, w)


def _tc_project(g, we, b, l):
    bb = 16
    rows = bb * l

    def mm(g_ref, we_ref, o_ref):
        acc = lax.dot_general(
            g_ref[:, :EMB],
            we_ref[...],
            (((1,), (1,)), ((), ())),
            preferred_element_type=jnp.float32,
        )
        o_ref[...] = acc.reshape(bb, l, HID)

    return pl.pallas_call(
        mm,
        grid=(b // bb,),
        in_specs=[
            pl.BlockSpec((rows, HID), lambda i: (i, 0)),
            pl.BlockSpec((HID, EMB), lambda i: (0, 0)),
        ],
        out_specs=pl.BlockSpec((bb, l, HID), lambda i: (i, 0, 0)),
        out_shape=jax.ShapeDtypeStruct((b, l, HID), jnp.float32),
    )(g, we)


def kernel(x, W, We):
    b, l = x.shape
    n = b * l
    x2d = x.reshape(n // _FIRE, _FIRE).astype(jnp.int32)
    g = _sc_gather(x2d, W)
    return _tc_project(g, We, b, l)
